# baseline (speedup 1.0000x reference)
import functools

import jax
import jax.numpy as jnp
from jax import lax
from jax.experimental import pallas as pl
from jax.experimental.pallas import tpu as pltpu
from jax.experimental.pallas import tpu_sc as plsc

N = 10000
R = 16
D = 128
E = 320000
B = 1024

NC = 2
NS = 16
Nh = N // NC
EpT = E // NS
KROWS = (EpT + 127) // 128
TAIL_FULL = (EpT % 128) // 16
EBLK = 1024
NFULL = EpT // EBLK
TAIL_E = EpT - NFULL * EBLK
TAIL_ROWS = TAIL_E // 128
GROUP = 4
KPAD = ((KROWS + 2 * GROUP - 1) // (2 * GROUP)) * (2 * GROUP)
NGRP = KPAD // GROUP
NCHUNK = D // 16
TRASH = R * Nh
ZROWS = 128
ASH_STRIPE = 5120
ASH_ROWS = NS * ASH_STRIPE

_mesh = plsc.VectorSubcoreMesh(core_axis_name="c", subcore_axis_name="s")
_sc_params = pltpu.CompilerParams(use_tc_tiling_on_sc=False)


def _make_sc_agg(do_counts):
  out_type = [jax.ShapeDtypeStruct((R, N, D), jnp.float32)]
  if do_counts:
    out_type.append(jax.ShapeDtypeStruct((R, N, 16), jnp.float32))

  scratch = [
      pltpu.VMEM((EBLK,), jnp.int32),
      pltpu.VMEM((EBLK,), jnp.int32),
      pltpu.VMEM((EBLK,), jnp.int32),
      pltpu.VMEM((KPAD, 128), jnp.int32),
      pltpu.VMEM((KPAD, 128), jnp.int32),
      pltpu.VMEM((128,), jnp.int32),
      pltpu.VMEM((128, 16), jnp.float32),
      pltpu.VMEM((ZROWS, 16), jnp.float32),
      pltpu.VMEM_SHARED((ASH_ROWS, 16), jnp.float32),
      pltpu.SemaphoreType.DMA,
      pltpu.SemaphoreType.DMA,
  ]

  def body(x2, e_src, e_dst, et, *rest):
    if do_counts:
      a_out, cnt_out = rest[0], rest[1]
      rest = rest[2:]
    else:
      a_out = rest[0]
      rest = rest[1:]
    (sblk, dblk, tblk, gb_v, sidx_v, gidx_b, rows_v, zbuf, a_sh,
     gsem, zsem) = rest

    c = lax.axis_index("c")
    s = lax.axis_index("s")
    base = s * EpT
    lo = c * Nh

    def fill_z(i, carry):
      zbuf[i, :] = jnp.zeros((16,), jnp.float32)
      return carry
    lax.fori_loop(0, ZROWS, fill_z, 0)

    def load_blk(k, nb):
      c1 = pltpu.async_copy(e_src.at[pl.ds(base + k * EBLK, nb)],
                            sblk.at[pl.ds(0, nb)], zsem)
      c2 = pltpu.async_copy(e_dst.at[pl.ds(base + k * EBLK, nb)],
                            dblk.at[pl.ds(0, nb)], zsem)
      c3 = pltpu.async_copy(et.at[pl.ds(base + k * EBLK, nb)],
                            tblk.at[pl.ds(0, nb)], zsem)
      c1.wait(); c2.wait(); c3.wait()

    def build_slice(e0):
      srcv = sblk[pl.ds(e0, 16)]
      dstv = dblk[pl.ds(e0, 16)]
      typv = tblk[pl.ds(e0, 16)]
      inh = (dstv >= lo) & (dstv < lo + Nh)
      sx = jnp.where(inh, typv * Nh + (dstv - lo), TRASH)
      return srcv * 8, sx

    def full_blk(k, carry):
      load_blk(k, EBLK)
      def brow(j2, carry2):
        for u in range(8):
          gb, sx = build_slice(j2 * 128 + u * 16)
          gb_v[k * (EBLK // 128) + j2, pl.ds(u * 16, 16)] = gb
          sidx_v[k * (EBLK // 128) + j2, pl.ds(u * 16, 16)] = sx
        return carry2
      return lax.fori_loop(0, EBLK // 128, brow, carry)
    lax.fori_loop(0, NFULL, full_blk, 0)

    load_blk(NFULL, TAIL_E)
    def trow(j2, carry2):
      for u in range(8):
        gb, sx = build_slice(j2 * 128 + u * 16)
        gb_v[NFULL * (EBLK // 128) + j2, pl.ds(u * 16, 16)] = gb
        sidx_v[NFULL * (EBLK // 128) + j2, pl.ds(u * 16, 16)] = sx
      return carry2
    lax.fori_loop(0, TAIL_ROWS, trow, 0)
    for u in range(8):
      if u < TAIL_FULL:
        gb, sx = build_slice(TAIL_ROWS * 128 + u * 16)
      else:
        gb = jnp.zeros((16,), jnp.int32)
        sx = jnp.full((16,), TRASH, jnp.int32)
      gb_v[KROWS - 1, pl.ds(u * 16, 16)] = gb
      sidx_v[KROWS - 1, pl.ds(u * 16, 16)] = sx
    nrows = KROWS

    def zero_ash():
      nz = ASH_STRIPE // ZROWS
      def zf(k, carry):
        pltpu.async_copy(
            zbuf, a_sh.at[pl.ds(s * ASH_STRIPE + k * ZROWS, ZROWS), :], zsem)
        return carry
      lax.fori_loop(0, nz, zf, 0)
      def zw(k, carry):
        pltpu.make_async_copy(
            zbuf, a_sh.at[pl.ds(s * ASH_STRIPE + k * ZROWS, ZROWS), :],
            zsem).wait()
        return carry
      lax.fori_loop(0, nz, zw, 0)

    if do_counts:
      def fill_o(i, carry):
        rows_v[i, :] = jnp.ones((16,), jnp.float32)
        return carry
      lax.fori_loop(0, 128, fill_o, 0)
      zero_ash()
      plsc.subcore_barrier()

      def cscat(j, carry):
        pltpu.sync_copy(rows_v, a_sh.at[sidx_v.at[j]], add=True)
        return carry
      lax.fori_loop(0, nrows, cscat, 0)
      plsc.subcore_barrier()
      pltpu.sync_copy(a_sh.at[pl.ds(s * Nh, Nh), :],
                      cnt_out.at[s, pl.ds(c * Nh, Nh), :])
      plsc.subcore_barrier()

    for dc in range(NCHUNK):
      zero_ash()
      plsc.subcore_barrier()

      if dc > 0:
        def bump(j, carry):
          for u in range(8):
            gb_v[j, pl.ds(u * 16, 16)] = gb_v[j, pl.ds(u * 16, 16)] + 1
          return carry
        lax.fori_loop(0, nrows, bump, 0)

      def chunk_row(j, carry):
        pltpu.async_copy(x2.at[gb_v.at[j]], rows_v, gsem).wait()
        pltpu.sync_copy(rows_v, a_sh.at[sidx_v.at[j]], add=True)
        return carry
      lax.fori_loop(0, nrows, chunk_row, 0)
      plsc.subcore_barrier()
      pltpu.sync_copy(a_sh.at[pl.ds(s * Nh, Nh), :],
                      a_out.at[s, pl.ds(c * Nh, Nh), pl.ds(dc * 16, 16)])
      plsc.subcore_barrier()

  return pl.kernel(body, out_type=tuple(out_type) if do_counts else out_type[0],
                   mesh=_mesh, scratch_types=scratch,
                   compiler_params=_sc_params)


_sc_agg_counts = _make_sc_agg(True)
_sc_agg = _make_sc_agg(False)


BpW = B // (NC * NS)


@functools.partial(
    pl.kernel,
    out_type=jax.ShapeDtypeStruct((B, D), jnp.float32),
    mesh=_mesh,
    scratch_types=[
        pltpu.VMEM((BpW,), jnp.int32),
        pltpu.VMEM((BpW, D), jnp.float32),
        pltpu.SemaphoreType.DMA,
    ],
    compiler_params=_sc_params,
)
def _sc_gather_rows(enr, idx, out, idx_v, rows_v, sem):
  wid = lax.axis_index("s") * NC + lax.axis_index("c")
  base = wid * BpW
  pltpu.sync_copy(idx.at[pl.ds(base, BpW)], idx_v)
  pltpu.async_copy(enr.at[idx_v], rows_v, sem).wait()
  pltpu.sync_copy(rows_v, out.at[pl.ds(base, BpW)])


BLK = 400
NBLK = N // BLK
_tc_params = pltpu.CompilerParams(dimension_semantics=("arbitrary",))


def _layer_acc(cnt_ref, a_ref, x_ref, w_ref, wr_ref, b_ref):
  acc = jnp.dot(x_ref[...], wr_ref[...], preferred_element_type=jnp.float32)
  acc = acc + b_ref[...]
  norm = 1.0 / jnp.maximum(cnt_ref[...], 1.0)
  for r in range(R):
    acc = acc + jnp.dot(a_ref[r] * norm[:, r:r + 1], w_ref[r],
                        preferred_element_type=jnp.float32)
  return acc


def _l1_body(cnt_ref, a_ref, x_ref, w_ref, wr_ref, b_ref,
             h_ref, sum_ref, sq_ref):
  acc = _layer_acc(cnt_ref, a_ref, x_ref, w_ref, wr_ref, b_ref)
  h_ref[...] = acc

  @pl.when(pl.program_id(0) == 0)
  def _():
    sum_ref[...] = jnp.zeros_like(sum_ref)
    sq_ref[...] = jnp.zeros_like(sq_ref)
  sum_ref[...] += jnp.sum(acc, axis=0, keepdims=True)
  sq_ref[...] += jnp.sum(acc * acc, axis=0, keepdims=True)


_layer_in_specs = [
    pl.BlockSpec((BLK, R), lambda i: (i, 0)),
    pl.BlockSpec((R, BLK, D), lambda i: (0, i, 0)),
    pl.BlockSpec((BLK, D), lambda i: (i, 0)),
    pl.BlockSpec((R, D, D), lambda i: (0, 0, 0)),
    pl.BlockSpec((D, D), lambda i: (0, 0)),
    pl.BlockSpec((1, D), lambda i: (0, 0)),
]

_tc_layer1 = pl.pallas_call(
    _l1_body,
    grid=(NBLK,),
    in_specs=_layer_in_specs,
    out_specs=[
        pl.BlockSpec((BLK, D), lambda i: (i, 0)),
        pl.BlockSpec((1, D), lambda i: (0, 0)),
        pl.BlockSpec((1, D), lambda i: (0, 0)),
    ],
    out_shape=[
        jax.ShapeDtypeStruct((N, D), jnp.float32),
        jax.ShapeDtypeStruct((1, D), jnp.float32),
        jax.ShapeDtypeStruct((1, D), jnp.float32),
    ],
    compiler_params=_tc_params,
)


def _bn_body(h_ref, sum_ref, sq_ref, g_ref, be_ref, out_ref):
  mu = sum_ref[...] / N
  var = sq_ref[...] / N - mu * mu
  inv = lax.rsqrt(var + 1e-5)
  out_ref[...] = jnp.maximum(
      g_ref[...] * (h_ref[...] - mu) * inv + be_ref[...], 0.0)


_tc_bn_relu = pl.pallas_call(
    _bn_body,
    grid=(NBLK,),
    in_specs=[
        pl.BlockSpec((BLK, D), lambda i: (i, 0)),
        pl.BlockSpec((1, D), lambda i: (0, 0)),
        pl.BlockSpec((1, D), lambda i: (0, 0)),
        pl.BlockSpec((1, D), lambda i: (0, 0)),
        pl.BlockSpec((1, D), lambda i: (0, 0)),
    ],
    out_specs=pl.BlockSpec((BLK, D), lambda i: (i, 0)),
    out_shape=jax.ShapeDtypeStruct((N, D), jnp.float32),
    compiler_params=_tc_params,
)


def _l2_body(cnt_ref, a_ref, h_ref, w_ref, wr_ref, b_ref, out_ref):
  acc = _layer_acc(cnt_ref, a_ref, h_ref, w_ref, wr_ref, b_ref)
  out_ref[...] = jnp.maximum(acc + h_ref[...], 0.0)


_tc_layer2 = pl.pallas_call(
    _l2_body,
    grid=(NBLK,),
    in_specs=_layer_in_specs,
    out_specs=pl.BlockSpec((BLK, D), lambda i: (i, 0)),
    out_shape=jax.ShapeDtypeStruct((N, D), jnp.float32),
    compiler_params=_tc_params,
)


BB = 128


def _dec_body(hv_ref, oh_ref, re_ref, enr_ref, out_ref):
  rv = jnp.dot(oh_ref[...], re_ref[...], preferred_element_type=jnp.float32)
  q = hv_ref[...] * rv
  out_ref[...] = lax.dot_general(q, enr_ref[...], (((1,), (1,)), ((), ())),
                                 preferred_element_type=jnp.float32)


_tc_decoder = pl.pallas_call(
    _dec_body,
    grid=(B // BB,),
    in_specs=[
        pl.BlockSpec((BB, D), lambda i: (i, 0)),
        pl.BlockSpec((BB, R), lambda i: (i, 0)),
        pl.BlockSpec((R, D), lambda i: (0, 0)),
        pl.BlockSpec((N, D), lambda i: (0, 0)),
    ],
    out_specs=pl.BlockSpec((BB, N), lambda i: (i, 0)),
    out_shape=jax.ShapeDtypeStruct((B, N), jnp.float32),
    compiler_params=_tc_params,
)


def kernel(h_idx, r_idx, edge_index, edge_type, entity_emb, rel_emb,
           W1, Wroot1, b1, gamma, beta, W2, Wroot2, b2):
  e_src = edge_index[0].astype(jnp.int32)
  e_dst = edge_index[1].astype(jnp.int32)
  et = edge_type.astype(jnp.int32)

  x2 = entity_emb.reshape(N * 8, 16)
  A1, counts16 = _sc_agg_counts(x2, e_src, e_dst, et)
  cnt_nr = counts16[:, :, 0].T

  H1, sums, sumsq = _tc_layer1(cnt_nr, A1, entity_emb, W1, Wroot1,
                               b1.reshape(1, D))
  h = _tc_bn_relu(H1, sums, sumsq, gamma.reshape(1, D), beta.reshape(1, D))

  A2 = _sc_agg(h.reshape(N * 8, 16), e_src, e_dst, et)
  enriched = _tc_layer2(cnt_nr, A2, h, W2, Wroot2, b2.reshape(1, D))

  hv = _sc_gather_rows(enriched, h_idx.astype(jnp.int32))
  oh = (r_idx.astype(jnp.int32)[:, None]
        == jnp.arange(R, dtype=jnp.int32)[None, :]).astype(jnp.float32)
  return _tc_decoder(hv, oh, rel_emb, enriched)


# speedup vs baseline: 1.0117x; 1.0117x over previous
import functools

import jax
import jax.numpy as jnp
from jax import lax
from jax.experimental import pallas as pl
from jax.experimental.pallas import tpu as pltpu
from jax.experimental.pallas import tpu_sc as plsc

N = 10000
R = 16
D = 128
E = 320000
B = 1024

NC = 2
NS = 16
Nh = N // NC
EpT = E // NS
KROWS = (EpT + 127) // 128
TAIL_FULL = (EpT % 128) // 16
EBLK = 1024
NFULL = EpT // EBLK
TAIL_E = EpT - NFULL * EBLK
TAIL_ROWS = TAIL_E // 128
GROUP = 4
KPAD = ((KROWS + 2 * GROUP - 1) // (2 * GROUP)) * (2 * GROUP)
NGRP = KPAD // GROUP
NCHUNK = D // 16
TRASH = R * Nh
ZROWS = 128
ASH_STRIPE = 5120
ASH_ROWS = NS * ASH_STRIPE

_mesh = plsc.VectorSubcoreMesh(core_axis_name="c", subcore_axis_name="s")
_sc_params = pltpu.CompilerParams(use_tc_tiling_on_sc=False)


def _make_sc_agg(do_counts):
  out_type = [jax.ShapeDtypeStruct((R, N, D), jnp.float32)]
  if do_counts:
    out_type.append(jax.ShapeDtypeStruct((R, N, 16), jnp.float32))

  scratch = [
      pltpu.VMEM((EBLK,), jnp.int32),
      pltpu.VMEM((EBLK,), jnp.int32),
      pltpu.VMEM((EBLK,), jnp.int32),
      pltpu.VMEM((KPAD, 128), jnp.int32),
      pltpu.VMEM((KPAD, 128), jnp.int32),
      pltpu.VMEM((128,), jnp.int32),
      pltpu.VMEM((128, 16), jnp.float32),
      pltpu.VMEM((ZROWS, 16), jnp.float32),
      pltpu.VMEM_SHARED((ASH_ROWS, 16), jnp.float32),
      pltpu.SemaphoreType.DMA,
      pltpu.SemaphoreType.DMA,
  ]

  def body(x2, e_src, e_dst, et, *rest):
    if do_counts:
      a_out, cnt_out = rest[0], rest[1]
      rest = rest[2:]
    else:
      a_out = rest[0]
      rest = rest[1:]
    (sblk, dblk, tblk, gb_v, sidx_v, gidx_b, rows_v, zbuf, a_sh,
     gsem, zsem) = rest

    c = lax.axis_index("c")
    s = lax.axis_index("s")
    base = s * EpT
    lo = c * Nh

    def fill_z(i, carry):
      zbuf[i, :] = jnp.zeros((16,), jnp.float32)
      return carry
    lax.fori_loop(0, ZROWS, fill_z, 0)

    def load_blk(k, nb):
      c1 = pltpu.async_copy(e_src.at[pl.ds(base + k * EBLK, nb)],
                            sblk.at[pl.ds(0, nb)], zsem)
      c2 = pltpu.async_copy(e_dst.at[pl.ds(base + k * EBLK, nb)],
                            dblk.at[pl.ds(0, nb)], zsem)
      c3 = pltpu.async_copy(et.at[pl.ds(base + k * EBLK, nb)],
                            tblk.at[pl.ds(0, nb)], zsem)
      c1.wait(); c2.wait(); c3.wait()

    def build_slice(e0):
      srcv = sblk[pl.ds(e0, 16)]
      dstv = dblk[pl.ds(e0, 16)]
      typv = tblk[pl.ds(e0, 16)]
      inh = (dstv >= lo) & (dstv < lo + Nh)
      sx = jnp.where(inh, typv * Nh + (dstv - lo), TRASH)
      return srcv * 8, sx

    def full_blk(k, carry):
      load_blk(k, EBLK)
      def brow(j2, carry2):
        for u in range(8):
          gb, sx = build_slice(j2 * 128 + u * 16)
          gb_v[k * (EBLK // 128) + j2, pl.ds(u * 16, 16)] = gb
          sidx_v[k * (EBLK // 128) + j2, pl.ds(u * 16, 16)] = sx
        return carry2
      return lax.fori_loop(0, EBLK // 128, brow, carry)
    lax.fori_loop(0, NFULL, full_blk, 0)

    load_blk(NFULL, TAIL_E)
    def trow(j2, carry2):
      for u in range(8):
        gb, sx = build_slice(j2 * 128 + u * 16)
        gb_v[NFULL * (EBLK // 128) + j2, pl.ds(u * 16, 16)] = gb
        sidx_v[NFULL * (EBLK // 128) + j2, pl.ds(u * 16, 16)] = sx
      return carry2
    lax.fori_loop(0, TAIL_ROWS, trow, 0)
    for u in range(8):
      if u < TAIL_FULL:
        gb, sx = build_slice(TAIL_ROWS * 128 + u * 16)
      else:
        gb = jnp.zeros((16,), jnp.int32)
        sx = jnp.full((16,), TRASH, jnp.int32)
      gb_v[KROWS - 1, pl.ds(u * 16, 16)] = gb
      sidx_v[KROWS - 1, pl.ds(u * 16, 16)] = sx
    nrows = KROWS

    def zero_ash():
      nz = ASH_STRIPE // ZROWS
      def zf(k, carry):
        pltpu.async_copy(
            zbuf, a_sh.at[pl.ds(s * ASH_STRIPE + k * ZROWS, ZROWS), :], zsem)
        return carry
      lax.fori_loop(0, nz, zf, 0)
      def zw(k, carry):
        pltpu.make_async_copy(
            zbuf, a_sh.at[pl.ds(s * ASH_STRIPE + k * ZROWS, ZROWS), :],
            zsem).wait()
        return carry
      lax.fori_loop(0, nz, zw, 0)

    if do_counts:
      def fill_o(i, carry):
        rows_v[i, :] = jnp.ones((16,), jnp.float32)
        return carry
      lax.fori_loop(0, 128, fill_o, 0)
      zero_ash()
      plsc.subcore_barrier()

      def cscat(j, carry):
        pltpu.sync_copy(rows_v, a_sh.at[sidx_v.at[j]], add=True)
        return carry
      lax.fori_loop(0, nrows, cscat, 0)
      plsc.subcore_barrier()
      pltpu.sync_copy(a_sh.at[pl.ds(s * Nh, Nh), :],
                      cnt_out.at[s, pl.ds(c * Nh, Nh), :])
      plsc.subcore_barrier()

    for dc in range(NCHUNK):
      zero_ash()
      plsc.subcore_barrier()

      def chunk_row(j, carry, dc=dc):
        for u in range(8):
          gidx_b[pl.ds(u * 16, 16)] = gb_v[j, pl.ds(u * 16, 16)] + dc
        pltpu.async_copy(x2.at[gidx_b], rows_v, gsem).wait()
        pltpu.sync_copy(rows_v, a_sh.at[sidx_v.at[j]], add=True)
        return carry
      lax.fori_loop(0, nrows, chunk_row, 0)
      plsc.subcore_barrier()
      pltpu.sync_copy(a_sh.at[pl.ds(s * Nh, Nh), :],
                      a_out.at[s, pl.ds(c * Nh, Nh), pl.ds(dc * 16, 16)])
      plsc.subcore_barrier()

  return pl.kernel(body, out_type=tuple(out_type) if do_counts else out_type[0],
                   mesh=_mesh, scratch_types=scratch,
                   compiler_params=_sc_params)


_sc_agg_counts = _make_sc_agg(True)
_sc_agg = _make_sc_agg(False)


BpW = B // (NC * NS)


@functools.partial(
    pl.kernel,
    out_type=jax.ShapeDtypeStruct((B, D), jnp.float32),
    mesh=_mesh,
    scratch_types=[
        pltpu.VMEM((BpW,), jnp.int32),
        pltpu.VMEM((BpW, D), jnp.float32),
        pltpu.SemaphoreType.DMA,
    ],
    compiler_params=_sc_params,
)
def _sc_gather_rows(enr, idx, out, idx_v, rows_v, sem):
  wid = lax.axis_index("s") * NC + lax.axis_index("c")
  base = wid * BpW
  pltpu.sync_copy(idx.at[pl.ds(base, BpW)], idx_v)
  pltpu.async_copy(enr.at[idx_v], rows_v, sem).wait()
  pltpu.sync_copy(rows_v, out.at[pl.ds(base, BpW)])


BLK = 400
NBLK = N // BLK
_tc_params = pltpu.CompilerParams(dimension_semantics=("arbitrary",))


def _layer_acc(cnt_ref, a_ref, x_ref, w_ref, wr_ref, b_ref):
  acc = jnp.dot(x_ref[...], wr_ref[...], preferred_element_type=jnp.float32)
  acc = acc + b_ref[...]
  norm = 1.0 / jnp.maximum(cnt_ref[...], 1.0)
  for r in range(R):
    acc = acc + jnp.dot(a_ref[r] * norm[:, r:r + 1], w_ref[r],
                        preferred_element_type=jnp.float32)
  return acc


def _l1_body(cnt_ref, a_ref, x_ref, w_ref, wr_ref, b_ref,
             h_ref, sum_ref, sq_ref):
  acc = _layer_acc(cnt_ref, a_ref, x_ref, w_ref, wr_ref, b_ref)
  h_ref[...] = acc

  @pl.when(pl.program_id(0) == 0)
  def _():
    sum_ref[...] = jnp.zeros_like(sum_ref)
    sq_ref[...] = jnp.zeros_like(sq_ref)
  sum_ref[...] += jnp.sum(acc, axis=0, keepdims=True)
  sq_ref[...] += jnp.sum(acc * acc, axis=0, keepdims=True)


_layer_in_specs = [
    pl.BlockSpec((BLK, R), lambda i: (i, 0)),
    pl.BlockSpec((R, BLK, D), lambda i: (0, i, 0)),
    pl.BlockSpec((BLK, D), lambda i: (i, 0)),
    pl.BlockSpec((R, D, D), lambda i: (0, 0, 0)),
    pl.BlockSpec((D, D), lambda i: (0, 0)),
    pl.BlockSpec((1, D), lambda i: (0, 0)),
]

_tc_layer1 = pl.pallas_call(
    _l1_body,
    grid=(NBLK,),
    in_specs=_layer_in_specs,
    out_specs=[
        pl.BlockSpec((BLK, D), lambda i: (i, 0)),
        pl.BlockSpec((1, D), lambda i: (0, 0)),
        pl.BlockSpec((1, D), lambda i: (0, 0)),
    ],
    out_shape=[
        jax.ShapeDtypeStruct((N, D), jnp.float32),
        jax.ShapeDtypeStruct((1, D), jnp.float32),
        jax.ShapeDtypeStruct((1, D), jnp.float32),
    ],
    compiler_params=_tc_params,
)


def _bn_body(h_ref, sum_ref, sq_ref, g_ref, be_ref, out_ref):
  mu = sum_ref[...] / N
  var = sq_ref[...] / N - mu * mu
  inv = lax.rsqrt(var + 1e-5)
  out_ref[...] = jnp.maximum(
      g_ref[...] * (h_ref[...] - mu) * inv + be_ref[...], 0.0)


_tc_bn_relu = pl.pallas_call(
    _bn_body,
    grid=(NBLK,),
    in_specs=[
        pl.BlockSpec((BLK, D), lambda i: (i, 0)),
        pl.BlockSpec((1, D), lambda i: (0, 0)),
        pl.BlockSpec((1, D), lambda i: (0, 0)),
        pl.BlockSpec((1, D), lambda i: (0, 0)),
        pl.BlockSpec((1, D), lambda i: (0, 0)),
    ],
    out_specs=pl.BlockSpec((BLK, D), lambda i: (i, 0)),
    out_shape=jax.ShapeDtypeStruct((N, D), jnp.float32),
    compiler_params=_tc_params,
)


def _l2_body(cnt_ref, a_ref, h_ref, w_ref, wr_ref, b_ref, out_ref):
  acc = _layer_acc(cnt_ref, a_ref, h_ref, w_ref, wr_ref, b_ref)
  out_ref[...] = jnp.maximum(acc + h_ref[...], 0.0)


_tc_layer2 = pl.pallas_call(
    _l2_body,
    grid=(NBLK,),
    in_specs=_layer_in_specs,
    out_specs=pl.BlockSpec((BLK, D), lambda i: (i, 0)),
    out_shape=jax.ShapeDtypeStruct((N, D), jnp.float32),
    compiler_params=_tc_params,
)


BB = 128


def _dec_body(hv_ref, oh_ref, re_ref, enr_ref, out_ref):
  rv = jnp.dot(oh_ref[...], re_ref[...], preferred_element_type=jnp.float32)
  q = hv_ref[...] * rv
  out_ref[...] = lax.dot_general(q, enr_ref[...], (((1,), (1,)), ((), ())),
                                 preferred_element_type=jnp.float32)


_tc_decoder = pl.pallas_call(
    _dec_body,
    grid=(B // BB,),
    in_specs=[
        pl.BlockSpec((BB, D), lambda i: (i, 0)),
        pl.BlockSpec((BB, R), lambda i: (i, 0)),
        pl.BlockSpec((R, D), lambda i: (0, 0)),
        pl.BlockSpec((N, D), lambda i: (0, 0)),
    ],
    out_specs=pl.BlockSpec((BB, N), lambda i: (i, 0)),
    out_shape=jax.ShapeDtypeStruct((B, N), jnp.float32),
    compiler_params=_tc_params,
)


def kernel(h_idx, r_idx, edge_index, edge_type, entity_emb, rel_emb,
           W1, Wroot1, b1, gamma, beta, W2, Wroot2, b2):
  e_src = edge_index[0].astype(jnp.int32)
  e_dst = edge_index[1].astype(jnp.int32)
  et = edge_type.astype(jnp.int32)

  x2 = entity_emb.reshape(N * 8, 16)
  A1, counts16 = _sc_agg_counts(x2, e_src, e_dst, et)
  cnt_nr = counts16[:, :, 0].T

  H1, sums, sumsq = _tc_layer1(cnt_nr, A1, entity_emb, W1, Wroot1,
                               b1.reshape(1, D))
  h = _tc_bn_relu(H1, sums, sumsq, gamma.reshape(1, D), beta.reshape(1, D))

  A2 = _sc_agg(h.reshape(N * 8, 16), e_src, e_dst, et)
  enriched = _tc_layer2(cnt_nr, A2, h, W2, Wroot2, b2.reshape(1, D))

  hv = _sc_gather_rows(enriched, h_idx.astype(jnp.int32))
  oh = (r_idx.astype(jnp.int32)[:, None]
        == jnp.arange(R, dtype=jnp.int32)[None, :]).astype(jnp.float32)
  return _tc_decoder(hv, oh, rel_emb, enriched)
